# baseline (speedup 1.0000x reference)
import functools

import jax
import jax.numpy as jnp
from jax import lax
from jax.experimental import pallas as pl
from jax.experimental.pallas import tpu as pltpu
from jax.experimental.pallas import tpu_sc as plsc

_NUM_CODES = 8192
_D = 64
_NTOK = 16 * 576
_BR = 256
_NCH = 8
_CH = _NUM_CODES // _NCH
_KC = 4
_NW = 32
_CHUNK = 96
_CPW = _NTOK // (_NW * _CHUNK)


def _main_body(ze_ref, cb_ref, zn2_ref, cn2_ref, q_ref):
    ze = ze_ref[...]
    dot = lax.dot_general(
        ze, cb_ref[...], (((1,), (1,)), ((), ())),
        preferred_element_type=jnp.float32)
    d2 = (zn2_ref[...] - 2.0 * dot) + cn2_ref[...]
    m = jnp.min(d2, axis=1, keepdims=True)
    e = (lax.bitcast_convert_type(m, jnp.int32) >> 23) & 0xFF
    ulp = lax.bitcast_convert_type((e - 7) << 23, jnp.float32)
    thr = m + 4.0 * ulp
    acc_v = jnp.full((_BR, 1), jnp.inf, jnp.float32)
    acc_i = jnp.zeros((_BR, 1), jnp.int32)
    iota = lax.broadcasted_iota(jnp.int32, (_BR, _CH), 1)
    for c in range(_NCH):
        seg = lax.slice(d2, (0, c * _CH), (_BR, (c + 1) * _CH))
        prev = jnp.full((_BR, 1), -1, jnp.int32)
        for _ in range(_KC):
            sel = (seg <= thr) & (iota > prev)
            idx = jnp.min(jnp.where(sel, iota, _CH), axis=1, keepdims=True)
            val = jnp.min(jnp.where(iota == idx, seg, jnp.inf),
                          axis=1, keepdims=True)
            accq = acc_v.astype(jnp.bfloat16).astype(jnp.float32)
            take = val < accq
            acc_v = jnp.where(take, val, acc_v)
            acc_i = jnp.where(take, idx + c * _CH, acc_i)
            prev = idx
    q_ref[...] = acc_i


def _tc_stage(ze, cb, zn2, cn2):
    return pl.pallas_call(
        _main_body,
        grid=(_NTOK // _BR,),
        in_specs=[pl.BlockSpec((_BR, _D), lambda i: (i, 0)),
                  pl.BlockSpec((_NUM_CODES, _D), lambda i: (0, 0)),
                  pl.BlockSpec((_BR, 1), lambda i: (i, 0)),
                  pl.BlockSpec((1, _NUM_CODES), lambda i: (0, 0))],
        out_specs=pl.BlockSpec((_BR, 1), lambda i: (i, 0)),
        out_shape=jax.ShapeDtypeStruct((_NTOK, 1), jnp.int32),
    )(ze, cb, zn2, cn2)


def _sc_gather(cbp, idx1d):
    mesh = plsc.VectorSubcoreMesh(core_axis_name="c", subcore_axis_name="s")
    per_w = _CPW * _CHUNK

    @functools.partial(
        pl.kernel, mesh=mesh,
        out_type=jax.ShapeDtypeStruct((_NTOK, 2 * _D), jnp.float32),
        scratch_types=[pltpu.VMEM((_CPW * _CHUNK,), jnp.int32),
                       pltpu.VMEM((_CPW * _CHUNK, 2 * _D), jnp.float32),
                       pltpu.SemaphoreType.DMA],
    )
    def g(cb_hbm, idx_hbm, out_hbm, idx_v, rows_v, sem):
        wid = lax.axis_index("s") * 2 + lax.axis_index("c")
        pltpu.sync_copy(idx_hbm.at[pl.ds(wid * per_w, per_w)], idx_v)
        copies = [
            pltpu.async_copy(cb_hbm.at[idx_v.at[pl.ds(j * _CHUNK, _CHUNK)]],
                             rows_v.at[pl.ds(j * _CHUNK, _CHUNK)], sem)
            for j in range(_CPW)
        ]
        for c in copies:
            c.wait()
        pltpu.sync_copy(rows_v, out_hbm.at[pl.ds(wid * per_w, per_w)])

    return g(cbp, idx1d)


def kernel(z, codebook_weight, bn_gamma, bn_beta):
    z = z.astype(jnp.float32)
    z_e = z * lax.rsqrt(jnp.mean(z * z, axis=-1, keepdims=True) + 1e-6)
    w = codebook_weight
    mean = jnp.mean(w, axis=0)
    var = jnp.var(w, axis=0)
    wn = (w - mean) * lax.rsqrt(var + 1e-5) * bn_gamma + bn_beta
    cb = wn * lax.rsqrt(jnp.mean(wn * wn, axis=-1, keepdims=True) + 1e-6)

    ze = z_e.reshape(-1, _D)
    zn2 = jnp.sum(ze * ze, axis=1, keepdims=True)
    cn2 = jnp.sum(cb * cb, axis=1)[None, :]

    q2 = _tc_stage(ze, cb, zn2, cn2)

    cbp = jnp.concatenate(
        [cb, jnp.zeros((_NUM_CODES, _D), jnp.float32)], axis=1)
    zq_flat = _sc_gather(cbp, q2.reshape(_NTOK))[:, :_D]

    z_q = zq_flat.reshape(z.shape)
    q = q2.reshape(z.shape[:-1])
    z_q_st = z_e + lax.stop_gradient(z_q - z_e)
    return (z_q_st, z_e, q, z_q)


# speedup vs baseline: 1.0468x; 1.0468x over previous
import functools

import jax
import jax.numpy as jnp
from jax import lax
from jax.experimental import pallas as pl
from jax.experimental.pallas import tpu as pltpu
from jax.experimental.pallas import tpu_sc as plsc

_NUM_CODES = 8192
_D = 64
_NTOK = 16 * 576
_BR = 512
_NCH = 8
_CH = _NUM_CODES // _NCH
_KC = 4
_NW = 32
_CHUNK = 96
_CPW = _NTOK // (_NW * _CHUNK)


def _main_body(ze_ref, cb_ref, zn2_ref, cn2_ref, q_ref):
    ze = ze_ref[...]
    dot = lax.dot_general(
        ze, cb_ref[...], (((1,), (1,)), ((), ())),
        preferred_element_type=jnp.float32)
    d2 = (zn2_ref[...] - 2.0 * dot) + cn2_ref[...]
    m = jnp.min(d2, axis=1, keepdims=True)
    e = (lax.bitcast_convert_type(m, jnp.int32) >> 23) & 0xFF
    ulp = lax.bitcast_convert_type((e - 7) << 23, jnp.float32)
    thr = m + 4.0 * ulp
    acc_v = jnp.full((_BR, 1), jnp.inf, jnp.float32)
    acc_i = jnp.zeros((_BR, 1), jnp.int32)
    iota = lax.broadcasted_iota(jnp.int32, (_BR, _CH), 1)
    for c in range(_NCH):
        seg = lax.slice(d2, (0, c * _CH), (_BR, (c + 1) * _CH))
        prev = jnp.full((_BR, 1), -1, jnp.int32)
        for _ in range(_KC):
            sel = (seg <= thr) & (iota > prev)
            idx = jnp.min(jnp.where(sel, iota, _CH), axis=1, keepdims=True)
            val = jnp.min(jnp.where(iota == idx, seg, jnp.inf),
                          axis=1, keepdims=True)
            accq = acc_v.astype(jnp.bfloat16).astype(jnp.float32)
            take = val < accq
            acc_v = jnp.where(take, val, acc_v)
            acc_i = jnp.where(take, idx + c * _CH, acc_i)
            prev = idx
    q_ref[...] = acc_i


def _tc_stage(ze, cb, zn2, cn2):
    return pl.pallas_call(
        _main_body,
        grid=(_NTOK // _BR,),
        in_specs=[pl.BlockSpec((_BR, _D), lambda i: (i, 0)),
                  pl.BlockSpec((_NUM_CODES, _D), lambda i: (0, 0)),
                  pl.BlockSpec((_BR, 1), lambda i: (i, 0)),
                  pl.BlockSpec((1, _NUM_CODES), lambda i: (0, 0))],
        out_specs=pl.BlockSpec((_BR, 1), lambda i: (i, 0)),
        out_shape=jax.ShapeDtypeStruct((_NTOK, 1), jnp.int32),
    )(ze, cb, zn2, cn2)


def _sc_gather(cbp, idx1d):
    mesh = plsc.VectorSubcoreMesh(core_axis_name="c", subcore_axis_name="s")
    per_w = _CPW * _CHUNK

    @functools.partial(
        pl.kernel, mesh=mesh,
        out_type=jax.ShapeDtypeStruct((_NTOK, 2 * _D), jnp.float32),
        scratch_types=[pltpu.VMEM((_CPW * _CHUNK,), jnp.int32),
                       pltpu.VMEM((_CPW * _CHUNK, 2 * _D), jnp.float32),
                       pltpu.SemaphoreType.DMA],
    )
    def g(cb_hbm, idx_hbm, out_hbm, idx_v, rows_v, sem):
        wid = lax.axis_index("s") * 2 + lax.axis_index("c")
        pltpu.sync_copy(idx_hbm.at[pl.ds(wid * per_w, per_w)], idx_v)
        copies = [
            pltpu.async_copy(cb_hbm.at[idx_v.at[pl.ds(j * _CHUNK, _CHUNK)]],
                             rows_v.at[pl.ds(j * _CHUNK, _CHUNK)], sem)
            for j in range(_CPW)
        ]
        for c in copies:
            c.wait()
        pltpu.sync_copy(rows_v, out_hbm.at[pl.ds(wid * per_w, per_w)])

    return g(cbp, idx1d)


def kernel(z, codebook_weight, bn_gamma, bn_beta):
    z = z.astype(jnp.float32)
    z_e = z * lax.rsqrt(jnp.mean(z * z, axis=-1, keepdims=True) + 1e-6)
    w = codebook_weight
    mean = jnp.mean(w, axis=0)
    var = jnp.var(w, axis=0)
    wn = (w - mean) * lax.rsqrt(var + 1e-5) * bn_gamma + bn_beta
    cb = wn * lax.rsqrt(jnp.mean(wn * wn, axis=-1, keepdims=True) + 1e-6)

    ze = z_e.reshape(-1, _D)
    zn2 = jnp.sum(ze * ze, axis=1, keepdims=True)
    cn2 = jnp.sum(cb * cb, axis=1)[None, :]

    q2 = _tc_stage(ze, cb, zn2, cn2)

    cbp = jnp.concatenate(
        [cb, jnp.zeros((_NUM_CODES, _D), jnp.float32)], axis=1)
    zq_flat = _sc_gather(cbp, q2.reshape(_NTOK))[:, :_D]

    z_q = zq_flat.reshape(z.shape)
    q = q2.reshape(z.shape[:-1])
    z_q_st = z_e + lax.stop_gradient(z_q - z_e)
    return (z_q_st, z_e, q, z_q)
